# Initial kernel scaffold; baseline (speedup 1.0000x reference)
import jax
import jax.numpy as jnp
from jax.experimental import pallas as pl


def kernel(feats, adj, W1_0, b1_0, W2_0, b2_0, g1_0, be1_0, g2_0, be2_0, W1_1, b1_1, W2_1, b2_1, g1_1, be1_1, g2_1, be2_1, W1_2, b1_2, W2_2, b2_2, g1_2, be1_2, g2_2, be2_2):
    raise NotImplementedError("write your pallas kernel here")


# speedup vs baseline: 1.0147x; 1.0147x over previous
import jax
import jax.numpy as jnp
from jax.experimental import pallas as pl

_N = 10000
_H = 128
_BI0 = 80
_BI = 400
_KC = 256

_DN = (((1,), (0,)), ((), ()))


def _dot(x, y):
    return jax.lax.dot_general(x, y, _DN, preferred_element_type=jnp.float32)


def _chunked_spmm(a, hv):
    z = None
    k = 0
    while k < _N:
        e = min(k + _KC, _N)
        p = _dot(a[:, k:e], hv[k:e, :])
        z = p if z is None else z + p
        k = e
    return z


def _spmm0_body(adj_ref, hbf_ref, z_ref, abf_ref):
    abf = adj_ref[...].astype(jnp.bfloat16)
    abf_ref[...] = abf
    z_ref[...] = _chunked_spmm(abf, hbf_ref[...])


def _spmm_body(abf_ref, hbf_ref, z_ref):
    z_ref[...] = _chunked_spmm(abf_ref[...], hbf_ref[...])


def _row_spec(blk, h):
    return pl.BlockSpec((blk, h), lambda i: (i, 0))


def _const_spec(shape):
    return pl.BlockSpec(shape, lambda i: (0, 0))


def _bn(x, g, b):
    mu = jnp.mean(x, axis=0)
    var = jnp.var(x, axis=0)
    return (x - mu) / jnp.sqrt(var + 1e-5) * g + b


def kernel(feats, adj,
           W1_0, b1_0, W2_0, b2_0, g1_0, be1_0, g2_0, be2_0,
           W1_1, b1_1, W2_1, b2_1, g1_1, be1_1, g2_1, be2_1,
           W1_2, b1_2, W2_2, b2_2, g1_2, be1_2, g2_2, be2_2):
    f32, bf16 = jnp.float32, jnp.bfloat16
    n0, n1 = _N // _BI0, _N // _BI

    params = [
        (W1_0, b1_0, W2_0, b2_0, g1_0, be1_0, g2_0, be2_0),
        (W1_1, b1_1, W2_1, b2_1, g1_1, be1_1, g2_1, be2_1),
        (W1_2, b1_2, W2_2, b2_2, g1_2, be1_2, g2_2, be2_2),
    ]

    h = feats
    abf = None
    for l in range(3):
        W1, b1, W2, b2, g1, be1, g2, be2 = params[l]
        hbf = h.astype(bf16)
        if l == 0:
            pooled, abf = pl.pallas_call(
                _spmm0_body, grid=(n0,),
                in_specs=[_row_spec(_BI0, _N), _const_spec((_N, _H))],
                out_specs=[_row_spec(_BI0, _H), _row_spec(_BI0, _N)],
                out_shape=[jax.ShapeDtypeStruct((_N, _H), f32),
                           jax.ShapeDtypeStruct((_N, _N), bf16)],
            )(adj, hbf)
        else:
            pooled = pl.pallas_call(
                _spmm_body, grid=(n1,),
                in_specs=[_row_spec(_BI, _N), _const_spec((_N, _H))],
                out_specs=_row_spec(_BI, _H),
                out_shape=jax.ShapeDtypeStruct((_N, _H), f32),
            )(abf, hbf)

        z = pooled @ W1 + b1
        z = jax.nn.relu(_bn(z, g1, be1))
        z = z @ W2 + b2
        h = jax.nn.relu(_bn(z, g2, be2))
    return h
